# Initial kernel scaffold; baseline (speedup 1.0000x reference)
import jax
import jax.numpy as jnp
from jax.experimental import pallas as pl


def kernel(features, adjacency_matrix, W_l, b_l, W_r):
    raise NotImplementedError("write your pallas kernel here")


# speedup vs baseline: 145.5408x; 145.5408x over previous
import jax
import jax.numpy as jnp
from jax.experimental import pallas as pl


def _sage_body(a_ref, x_ref, wl_ref, bl_ref, wr_ref, o_ref):
    a = a_ref[...]
    x = x_ref[...]
    num = jax.lax.dot_general(
        a, x, (((0,), (0,)), ((), ())),
        preferred_element_type=jnp.float32,
        precision=jax.lax.Precision.HIGHEST)
    ones = jnp.ones((a.shape[0], 1), dtype=jnp.float32)
    cnt = jax.lax.dot_general(
        a, ones, (((0,), (0,)), ((), ())),
        preferred_element_type=jnp.float32,
        precision=jax.lax.Precision.HIGHEST)
    agg = num / jnp.maximum(cnt, 1.0)
    h = jax.lax.dot_general(
        agg, wl_ref[...], (((1,), (1,)), ((), ())),
        preferred_element_type=jnp.float32,
        precision=jax.lax.Precision.HIGHEST)
    h = h + bl_ref[...]
    h = h + jax.lax.dot_general(
        x, wr_ref[...], (((1,), (1,)), ((), ())),
        preferred_element_type=jnp.float32,
        precision=jax.lax.Precision.HIGHEST)
    o_ref[...] = h


def kernel(features, adjacency_matrix, W_l, b_l, W_r):
    n, d = features.shape
    return pl.pallas_call(
        _sage_body,
        out_shape=jax.ShapeDtypeStruct((n, d), jnp.float32),
    )(adjacency_matrix, features, W_l, b_l.reshape(1, d), W_r)


# speedup vs baseline: 241.3586x; 1.6584x over previous
import jax
import jax.numpy as jnp
from jax.experimental import pallas as pl


def _sage_body(a_ref, x_ref, wl_ref, bl_ref, wr_ref, o_ref):
    a = a_ref[...].astype(jnp.bfloat16)
    x = x_ref[...]
    x_hi = x.astype(jnp.bfloat16)
    x_lo = (x - x_hi.astype(jnp.float32)).astype(jnp.bfloat16)
    dn = (((0,), (0,)), ((), ()))
    num = (jax.lax.dot_general(a, x_hi, dn, preferred_element_type=jnp.float32)
           + jax.lax.dot_general(a, x_lo, dn, preferred_element_type=jnp.float32))
    ones = jnp.ones((a.shape[0], 1), dtype=jnp.bfloat16)
    cnt = jax.lax.dot_general(a, ones, dn, preferred_element_type=jnp.float32)
    agg = num / jnp.maximum(cnt, 1.0)
    dt = (((1,), (1,)), ((), ()))
    h = jax.lax.dot_general(
        agg, wl_ref[...], dt,
        preferred_element_type=jnp.float32,
        precision=jax.lax.Precision.HIGHEST)
    h = h + bl_ref[...]
    h = h + jax.lax.dot_general(
        x, wr_ref[...], dt,
        preferred_element_type=jnp.float32,
        precision=jax.lax.Precision.HIGHEST)
    o_ref[...] = h


def kernel(features, adjacency_matrix, W_l, b_l, W_r):
    n, d = features.shape
    return pl.pallas_call(
        _sage_body,
        out_shape=jax.ShapeDtypeStruct((n, d), jnp.float32),
    )(adjacency_matrix, features, W_l, b_l.reshape(1, d), W_r)
